# Initial kernel scaffold; baseline (speedup 1.0000x reference)
import jax
import jax.numpy as jnp
from jax.experimental import pallas as pl


def kernel(user_emb, item_emb, edge_vals, edge_index):
    raise NotImplementedError("write your pallas kernel here")


# speedup vs baseline: 3.1744x; 3.1744x over previous
import functools

import jax
import jax.numpy as jnp
from jax import lax
from jax.experimental import pallas as pl
from jax.experimental.pallas import tpu as pltpu
from jax.experimental.pallas import tpu_sc as plsc

N_USERS = 25000
N_ITEMS = 25000
N_NODES = N_USERS + N_ITEMS
N_EDGES = 800000
D = 64
H = 32
N_LAYERS = 3

NC = 2
NS = 16
C = 128

NEP = ((N_EDGES + NS * C - 1) // (NS * C)) * (NS * C)
EPT = NEP // NS
G = EPT // C

ZCH = 400
ZNCH = N_NODES // ZCH
ZK = (ZNCH + NS - 1) // NS


def _build_layer():
    mesh = plsc.VectorSubcoreMesh(core_axis_name="c", subcore_axis_name="s")

    @functools.partial(
        pl.kernel,
        out_type=jax.ShapeDtypeStruct((2 * N_NODES, H), jnp.float32),
        mesh=mesh,
        scratch_types=[
            pltpu.VMEM((C,), jnp.int32),
            pltpu.VMEM((C,), jnp.int32),
            pltpu.VMEM((C,), jnp.float32),
            pltpu.VMEM((C, H), jnp.float32),
            pltpu.VMEM((ZCH, H), jnp.float32),
            pltpu.VMEM_SHARED((N_NODES, H), jnp.float32),
        ],
        compiler_params=pltpu.CompilerParams(use_tc_tiling_on_sc=False),
    )
    def layer(xcat, srcp, dstp, valp, out, srcv, dstv, valv, rows, zbuf, acc):
        c = lax.axis_index("c")
        s = lax.axis_index("s")
        off = c * N_NODES

        def zb(i, carry):
            zbuf[i, pl.ds(0, 16)] = jnp.zeros((16,), jnp.float32)
            zbuf[i, pl.ds(16, 16)] = jnp.zeros((16,), jnp.float32)
            return carry

        lax.fori_loop(0, ZCH, zb, 0)

        def za(k, carry):
            idx = s + k * NS

            @pl.when(idx < ZNCH)
            def _():
                pltpu.sync_copy(zbuf, acc.at[pl.ds(idx * ZCH, ZCH)])

            return carry

        lax.fori_loop(0, ZK, za, 0)
        plsc.subcore_barrier()

        def body(g, carry):
            base = s * EPT + g * C
            pltpu.sync_copy(srcp.at[pl.ds(base, C)], srcv)
            pltpu.sync_copy(dstp.at[pl.ds(base, C)], dstv)
            pltpu.sync_copy(valp.at[pl.ds(base, C)], valv)

            def addoff(i, carry2):
                srcv[pl.ds(i * 16, 16)] = srcv[pl.ds(i * 16, 16)] + off
                return carry2

            lax.fori_loop(0, C // 16, addoff, 0)
            pltpu.sync_copy(xcat.at[srcv], rows)

            def scale(g2, carry2):
                v16 = valv[pl.ds(g2 * 16, 16)]
                for l in range(16):
                    v = v16[l]
                    e = g2 * 16 + l
                    rows[e, pl.ds(0, 16)] = rows[e, pl.ds(0, 16)] * v
                    rows[e, pl.ds(16, 16)] = rows[e, pl.ds(16, 16)] * v
                return carry2

            lax.fori_loop(0, C // 16, scale, 0)
            pltpu.sync_copy(rows, acc.at[dstv], add=True)
            return carry

        lax.fori_loop(0, G, body, 0)
        plsc.subcore_barrier()

        def wb(k, carry):
            idx = s + k * NS

            @pl.when(idx < ZNCH)
            def _():
                r0 = idx * ZCH
                pltpu.sync_copy(acc.at[pl.ds(r0, ZCH)], out.at[pl.ds(off + r0, ZCH)])

            return carry

        lax.fori_loop(0, ZK, wb, 0)

    return layer


_LAYER = _build_layer()

_MROWS = 2 * N_NODES * H // 128
_MBLK = 1000
_MGRID = _MROWS // _MBLK


def _mean_body(a, b, c, d, o):
    o[...] = (a[...] + b[...] + c[...] + d[...]) * 0.25


_MEAN = pl.pallas_call(
    _mean_body,
    out_shape=jax.ShapeDtypeStruct((_MROWS, 128), jnp.float32),
    grid=(_MGRID,),
    in_specs=[pl.BlockSpec((_MBLK, 128), lambda i: (i, 0))] * 4,
    out_specs=pl.BlockSpec((_MBLK, 128), lambda i: (i, 0)),
)


def kernel(user_emb, item_emb, edge_vals, edge_index):
    x0 = jnp.concatenate([user_emb, item_emb], axis=0)
    x0cat = jnp.concatenate([x0[:, :H], x0[:, H:]], axis=0)
    pad = NEP - N_EDGES
    srcp = jnp.concatenate([edge_index[0], jnp.zeros((pad,), jnp.int32)])
    dstp = jnp.concatenate([edge_index[1], jnp.zeros((pad,), jnp.int32)])
    valp = jnp.concatenate([edge_vals, jnp.zeros((pad,), jnp.float32)])

    x1 = _LAYER(x0cat, srcp, dstp, valp)
    x2 = _LAYER(x1, srcp, dstp, valp)
    x3 = _LAYER(x2, srcp, dstp, valp)

    m = _MEAN(
        x0cat.reshape(_MROWS, 128),
        x1.reshape(_MROWS, 128),
        x2.reshape(_MROWS, 128),
        x3.reshape(_MROWS, 128),
    ).reshape(2 * N_NODES, H)
    out = jnp.concatenate([m[:N_NODES], m[N_NODES:]], axis=1)
    return (out[:N_USERS], out[N_USERS:])


# speedup vs baseline: 6.4695x; 2.0380x over previous
import functools

import jax
import jax.numpy as jnp
from jax import lax
from jax.experimental import pallas as pl
from jax.experimental.pallas import tpu as pltpu
from jax.experimental.pallas import tpu_sc as plsc

N_USERS = 25000
N_ITEMS = 25000
N_NODES = N_USERS + N_ITEMS
N_EDGES = 800000
D = 64
H = 32
N_LAYERS = 3

NC = 2
NS = 16
C = 128
SLAB = 4

NEP = ((N_EDGES + NS * C * SLAB - 1) // (NS * C * SLAB)) * (NS * C * SLAB)
NCHUNKS = NEP // C
G = NEP // (NS * C)
NSLAB = G // SLAB

ZCH = 400
ZNCH = N_NODES // ZCH
ZK = (ZNCH + NS - 1) // NS
ZB = 80


def _build_layer():
    mesh = plsc.VectorSubcoreMesh(core_axis_name="c", subcore_axis_name="s")

    @functools.partial(
        pl.kernel,
        out_type=jax.ShapeDtypeStruct((2 * N_NODES, H), jnp.float32),
        mesh=mesh,
        scratch_types=[
            pltpu.VMEM((SLAB, C), jnp.int32),
            pltpu.VMEM((SLAB, C), jnp.int32),
            pltpu.VMEM((SLAB, C), jnp.float32),
            pltpu.VMEM((SLAB, C, H), jnp.float32),
            pltpu.VMEM((ZB, H), jnp.float32),
            pltpu.VMEM_SHARED((N_NODES, H), jnp.float32),
        ] + [pltpu.SemaphoreType.DMA] * SLAB,
        compiler_params=pltpu.CompilerParams(use_tc_tiling_on_sc=False),
    )
    def layer(xcat, srcs, dstp, valp, out, src_slab, dst_slab, val_slab,
              rows, zbuf, acc, *sems):
        c = lax.axis_index("c")
        s = lax.axis_index("s")
        off = c * N_NODES

        def zb(i, carry):
            zbuf[i, pl.ds(0, 16)] = jnp.zeros((16,), jnp.float32)
            zbuf[i, pl.ds(16, 16)] = jnp.zeros((16,), jnp.float32)
            return carry

        lax.fori_loop(0, ZB, zb, 0)

        def za(k, carry):
            idx = s + k * NS

            @pl.when(idx < ZNCH)
            def _():
                for i in range(ZCH // ZB):
                    pltpu.sync_copy(zbuf, acc.at[pl.ds(idx * ZCH + i * ZB, ZB)])

            return carry

        lax.fori_loop(0, ZK, za, 0)
        plsc.subcore_barrier()

        def body(g, carry):
            row0 = s * G + g * SLAB

            pltpu.sync_copy(srcs.at[c, pl.ds(row0, SLAB)], src_slab)
            pltpu.sync_copy(dstp.at[pl.ds(row0, SLAB)], dst_slab)
            pltpu.sync_copy(valp.at[pl.ds(row0, SLAB)], val_slab)

            descs = [
                pltpu.async_copy(xcat.at[src_slab.at[j]], rows.at[j], sems[j])
                for j in range(SLAB)
            ]
            for j in range(SLAB):
                descs[j].wait()

                def scale(g2, carry2, j=j):
                    v16 = val_slab[j, pl.ds(g2 * 16, 16)]
                    for l in range(16):
                        v = v16[l]
                        e = g2 * 16 + l
                        rows[j, e, pl.ds(0, 16)] = rows[j, e, pl.ds(0, 16)] * v
                        rows[j, e, pl.ds(16, 16)] = rows[j, e, pl.ds(16, 16)] * v
                    return carry2

                lax.fori_loop(0, C // 16, scale, 0)
                pltpu.sync_copy(rows.at[j], acc.at[dst_slab.at[j]], add=True)
            return carry

        lax.fori_loop(0, NSLAB, body, 0)
        plsc.subcore_barrier()

        def wb(k, carry):
            idx = s + k * NS

            @pl.when(idx < ZNCH)
            def _():
                r0 = idx * ZCH
                pltpu.sync_copy(acc.at[pl.ds(r0, ZCH)], out.at[pl.ds(off + r0, ZCH)])

            return carry

        lax.fori_loop(0, ZK, wb, 0)

    return layer


_LAYER = _build_layer()

_MROWS = 2 * N_NODES * H // 128
_MBLK = 1000
_MGRID = _MROWS // _MBLK


def _mean_body(a, b, c, d, o):
    o[...] = (a[...] + b[...] + c[...] + d[...]) * 0.25


_MEAN = pl.pallas_call(
    _mean_body,
    out_shape=jax.ShapeDtypeStruct((_MROWS, 128), jnp.float32),
    grid=(_MGRID,),
    in_specs=[pl.BlockSpec((_MBLK, 128), lambda i: (i, 0))] * 4,
    out_specs=pl.BlockSpec((_MBLK, 128), lambda i: (i, 0)),
)


def kernel(user_emb, item_emb, edge_vals, edge_index):
    x0 = jnp.concatenate([user_emb, item_emb], axis=0)
    x0cat = jnp.concatenate([x0[:, :H], x0[:, H:]], axis=0)
    pad = NEP - N_EDGES
    src = jnp.concatenate([edge_index[0], jnp.zeros((pad,), jnp.int32)])
    src0 = src.reshape(NCHUNKS, C)
    srcs = jnp.stack([src0, src0 + N_NODES])
    dstp = jnp.concatenate([edge_index[1], jnp.zeros((pad,), jnp.int32)]).reshape(NCHUNKS, C)
    valp = jnp.concatenate([edge_vals, jnp.zeros((pad,), jnp.float32)]).reshape(NCHUNKS, C)

    x1 = _LAYER(x0cat, srcs, dstp, valp)
    x2 = _LAYER(x1, srcs, dstp, valp)
    x3 = _LAYER(x2, srcs, dstp, valp)

    m = _MEAN(
        x0cat.reshape(_MROWS, 128),
        x1.reshape(_MROWS, 128),
        x2.reshape(_MROWS, 128),
        x3.reshape(_MROWS, 128),
    ).reshape(2 * N_NODES, H)
    out = jnp.concatenate([m[:N_NODES], m[N_NODES:]], axis=1)
    return (out[:N_USERS], out[N_USERS:])


# speedup vs baseline: 10.1107x; 1.5628x over previous
import functools

import jax
import jax.numpy as jnp
from jax import lax
from jax.experimental import pallas as pl
from jax.experimental.pallas import tpu as pltpu
from jax.experimental.pallas import tpu_sc as plsc

N_USERS = 25000
N_ITEMS = 25000
N_NODES = N_USERS + N_ITEMS
N_EDGES = 800000
D = 64
H = 32
N_LAYERS = 3

NC = 2
NS = 16
C = 128
SLAB = 4

NEP = ((N_EDGES + NS * C * SLAB - 1) // (NS * C * SLAB)) * (NS * C * SLAB)
NCHUNKS = NEP // C
G = NEP // (NS * C)
NSLAB = G // SLAB

ZCH = 400
ZNCH = N_NODES // ZCH
ZK = (ZNCH + NS - 1) // NS
ZB = 80


def _build_layer():
    mesh = plsc.VectorSubcoreMesh(core_axis_name="c", subcore_axis_name="s")

    @functools.partial(
        pl.kernel,
        out_type=jax.ShapeDtypeStruct((2 * N_NODES, H), jnp.float32),
        mesh=mesh,
        scratch_types=[
            pltpu.VMEM((2, SLAB, C), jnp.int32),
            pltpu.VMEM((2, SLAB, C), jnp.int32),
            pltpu.VMEM((2, SLAB, C), jnp.float32),
            pltpu.VMEM((SLAB, C, H), jnp.float32),
            pltpu.VMEM((ZB, H), jnp.float32),
            pltpu.VMEM_SHARED((N_NODES, H), jnp.float32),
        ] + [pltpu.SemaphoreType.DMA] * (SLAB + 3),
        compiler_params=pltpu.CompilerParams(use_tc_tiling_on_sc=False),
    )
    def layer(xcat, srcs, dstp, valp, out, src_slab, dst_slab, val_slab,
              rows, zbuf, acc, *sems):
        c = lax.axis_index("c")
        s = lax.axis_index("s")
        off = c * N_NODES

        def zb(i, carry):
            zbuf[i, pl.ds(0, 16)] = jnp.zeros((16,), jnp.float32)
            zbuf[i, pl.ds(16, 16)] = jnp.zeros((16,), jnp.float32)
            return carry

        lax.fori_loop(0, ZB, zb, 0)

        def za(k, carry):
            idx = s + k * NS

            @pl.when(idx < ZNCH)
            def _():
                for i in range(ZCH // ZB):
                    pltpu.sync_copy(zbuf, acc.at[pl.ds(idx * ZCH + i * ZB, ZB)])

            return carry

        lax.fori_loop(0, ZK, za, 0)
        plsc.subcore_barrier()

        gsems = sems[:SLAB]
        isems = sems[SLAB:SLAB + 2]
        scsem = sems[SLAB + 2]

        def fire_idx(g, k):
            rown = s * G + g * SLAB
            pltpu.async_copy(srcs.at[c, pl.ds(rown, SLAB)], src_slab.at[k], isems[k])
            pltpu.async_copy(dstp.at[pl.ds(rown, SLAB)], dst_slab.at[k], isems[k])
            pltpu.async_copy(valp.at[pl.ds(rown, SLAB)], val_slab.at[k], isems[k])

        def wait_idx(g, k):
            rown = s * G + g * SLAB
            pltpu.make_async_copy(srcs.at[c, pl.ds(rown, SLAB)], src_slab.at[k], isems[k]).wait()
            pltpu.make_async_copy(dstp.at[pl.ds(rown, SLAB)], dst_slab.at[k], isems[k]).wait()
            pltpu.make_async_copy(valp.at[pl.ds(rown, SLAB)], val_slab.at[k], isems[k]).wait()

        fire_idx(0, 0)

        def body(t, carry):
            for k in (0, 1):
                g = t * 2 + k
                wait_idx(g, k)

                @pl.when(g + 1 < NSLAB)
                def _(g=g, k=k):
                    fire_idx(g + 1, 1 - k)

                gd = [
                    pltpu.async_copy(xcat.at[src_slab.at[k, j]], rows.at[j], gsems[j])
                    for j in range(SLAB)
                ]
                sd = []
                for j in range(SLAB):
                    gd[j].wait()

                    def scale(g2, carry2, j=j, k=k):
                        v16 = val_slab[k, j, pl.ds(g2 * 16, 16)]
                        for l in range(16):
                            v = v16[l]
                            e = g2 * 16 + l
                            rows[j, e, pl.ds(0, 16)] = rows[j, e, pl.ds(0, 16)] * v
                            rows[j, e, pl.ds(16, 16)] = rows[j, e, pl.ds(16, 16)] * v
                        return carry2

                    lax.fori_loop(0, C // 16, scale, 0)
                    sd.append(pltpu.async_copy(rows.at[j], acc.at[dst_slab.at[k, j]],
                                               scsem, add=True))
                for d in sd:
                    d.wait()
            return carry

        assert NSLAB % 2 == 0
        lax.fori_loop(0, NSLAB // 2, body, 0)
        plsc.subcore_barrier()

        def wb(k, carry):
            idx = s + k * NS

            @pl.when(idx < ZNCH)
            def _():
                r0 = idx * ZCH
                pltpu.sync_copy(acc.at[pl.ds(r0, ZCH)], out.at[pl.ds(off + r0, ZCH)])

            return carry

        lax.fori_loop(0, ZK, wb, 0)

    return layer


_LAYER = _build_layer()

_MROWS = 2 * N_NODES * H // 128
_MBLK = 1000
_MGRID = _MROWS // _MBLK


def _mean_body(a, b, c, d, o):
    o[...] = (a[...] + b[...] + c[...] + d[...]) * 0.25


_MEAN = pl.pallas_call(
    _mean_body,
    out_shape=jax.ShapeDtypeStruct((_MROWS, 128), jnp.float32),
    grid=(_MGRID,),
    in_specs=[pl.BlockSpec((_MBLK, 128), lambda i: (i, 0))] * 4,
    out_specs=pl.BlockSpec((_MBLK, 128), lambda i: (i, 0)),
)


def kernel(user_emb, item_emb, edge_vals, edge_index):
    x0 = jnp.concatenate([user_emb, item_emb], axis=0)
    x0cat = jnp.concatenate([x0[:, :H], x0[:, H:]], axis=0)
    pad = NEP - N_EDGES
    src = jnp.concatenate([edge_index[0], jnp.zeros((pad,), jnp.int32)])
    src0 = src.reshape(NCHUNKS, C)
    srcs = jnp.stack([src0, src0 + N_NODES])
    dstp = jnp.concatenate([edge_index[1], jnp.zeros((pad,), jnp.int32)]).reshape(NCHUNKS, C)
    valp = jnp.concatenate([edge_vals, jnp.zeros((pad,), jnp.float32)]).reshape(NCHUNKS, C)

    x1 = _LAYER(x0cat, srcs, dstp, valp)
    x2 = _LAYER(x1, srcs, dstp, valp)
    x3 = _LAYER(x2, srcs, dstp, valp)

    m = _MEAN(
        x0cat.reshape(_MROWS, 128),
        x1.reshape(_MROWS, 128),
        x2.reshape(_MROWS, 128),
        x3.reshape(_MROWS, 128),
    ).reshape(2 * N_NODES, H)
    out = jnp.concatenate([m[:N_NODES], m[N_NODES:]], axis=1)
    return (out[:N_USERS], out[N_USERS:])
